# baseline (speedup 1.0000x reference)
import jax
import jax.numpy as jnp
from jax.experimental import pallas as pl
from jax.experimental.pallas import tpu as pltpu

E = 320000
D = 16
BLK = 160000


def _mlp_kernel(a_ref, wet_ref, be_ref, wft_ref, bf_ref, out_ref):
    h = jnp.maximum(
        jnp.dot(wet_ref[...], a_ref[...], preferred_element_type=jnp.float32)
        + be_ref[...],
        0.0,
    )
    out_ref[...] = (
        jnp.dot(wft_ref[...], h, preferred_element_type=jnp.float32)
        + bf_ref[0, 0]
    )


def kernel(x, adjs, edge_attr, Wn, bn, We, be, Wf, bf):
    at = edge_attr.astype(jnp.float32).T
    at = pltpu.with_memory_space_constraint(at, pltpu.MemorySpace.HBM)
    wet = We.astype(jnp.float32).T
    be2 = be.astype(jnp.float32).reshape(D, 1)
    wft = Wf.astype(jnp.float32).T
    bf2 = jnp.reshape(bf.astype(jnp.float32), (1, 1))

    out = pl.pallas_call(
        _mlp_kernel,
        grid=(E // BLK,),
        in_specs=[
            pl.BlockSpec((D, BLK), lambda i: (0, i)),
            pl.BlockSpec((D, D), lambda i: (0, 0)),
            pl.BlockSpec((D, 1), lambda i: (0, 0)),
            pl.BlockSpec((1, D), lambda i: (0, 0)),
            pl.BlockSpec((1, 1), lambda i: (0, 0)),
        ],
        out_specs=pl.BlockSpec((1, BLK), lambda i: (0, i)),
        out_shape=jax.ShapeDtypeStruct((1, E), jnp.float32),
    )(at, wet, be2, wft, bf2)

    return jnp.reshape(out, (E,))
